# Initial kernel scaffold; baseline (speedup 1.0000x reference)
import jax
import jax.numpy as jnp
from jax.experimental import pallas as pl


def kernel(label, feat, centers):
    raise NotImplementedError("write your pallas kernel here")


# speedup vs baseline: 3.7473x; 3.7473x over previous
import jax
import jax.numpy as jnp
from jax.experimental import pallas as pl
from jax.experimental.pallas import tpu as pltpu

NCLS = 100
FDIM = 512
BATCH_ = 4096
BLK = 512
LAMDA_ = 0.5
EPS = 1e-8


def _body(label_ref, feat_ref, centers_ref, out_ref, s_acc, cnt_acc, a_acc):
    i = pl.program_id(0)
    nsteps = pl.num_programs(0)

    @pl.when(i == 0)
    def _init():
        s_acc[...] = jnp.zeros_like(s_acc)
        cnt_acc[...] = jnp.zeros_like(cnt_acc)
        a_acc[...] = jnp.zeros_like(a_acc)

    feat = feat_ref[...]
    lbl = label_ref[0, 0, :]
    onehot = (lbl[:, None] == jax.lax.broadcasted_iota(jnp.int32, (1, NCLS), 1)
              ).astype(jnp.float32)
    s_acc[...] += jax.lax.dot_general(
        onehot, feat, (((0,), (0,)), ((), ())),
        preferred_element_type=jnp.float32)
    cnt_acc[...] += jnp.sum(onehot, axis=0, keepdims=True)
    a_acc[...] += jnp.sum(feat * feat, axis=0, keepdims=True)

    @pl.when(i == nsteps - 1)
    def _fini():
        centers = centers_ref[...]
        a = jnp.sum(a_acc[...])
        b = jnp.sum(s_acc[...] * centers)
        n2 = jnp.sum(centers * centers, axis=1)
        c = jnp.sum(cnt_acc[0, :] * n2)
        center_loss = (a - 2.0 * b + c) / 2.0 / BATCH_
        inv = 1.0 / jnp.maximum(jnp.sqrt(n2), EPS)
        cn = centers * inv[:, None]
        s_vec = jnp.sum(cn, axis=0)
        island = (jnp.sum(s_vec * s_vec) - jnp.sum(cn * cn)
                  + float(NCLS * NCLS - NCLS))
        out_ref[...] = jnp.reshape(center_loss + LAMDA_ * island, (1, 1))


def kernel(label, feat, centers):
    nblk = BATCH_ // BLK
    label3 = label.reshape(nblk, 1, BLK)
    out = pl.pallas_call(
        _body,
        grid=(nblk,),
        in_specs=[
            pl.BlockSpec((1, 1, BLK), lambda i: (i, 0, 0)),
            pl.BlockSpec((BLK, FDIM), lambda i: (i, 0)),
            pl.BlockSpec((NCLS, FDIM), lambda i: (0, 0)),
        ],
        out_specs=pl.BlockSpec((1, 1), lambda i: (0, 0)),
        out_shape=jax.ShapeDtypeStruct((1, 1), jnp.float32),
        scratch_shapes=[
            pltpu.VMEM((NCLS, FDIM), jnp.float32),
            pltpu.VMEM((1, NCLS), jnp.float32),
            pltpu.VMEM((1, FDIM), jnp.float32),
        ],
    )(label3, feat, centers)
    return out.reshape(1)
